# Initial kernel scaffold; baseline (speedup 1.0000x reference)
import jax
import jax.numpy as jnp
from jax.experimental import pallas as pl


def kernel(quant_pred, target_wav, ze, emb):
    raise NotImplementedError("write your pallas kernel here")


# speedup vs baseline: 1.6979x; 1.6979x over previous
import functools

import jax
import jax.numpy as jnp
from jax import lax
from jax.experimental import pallas as pl
from jax.experimental.pallas import tpu as pltpu
from jax.experimental.pallas import tpu_sc as plsc

_BETA = 0.25
_B, _Q, _K, _C, _T = 8, 64, 512, 256, 2048
_TB = 256
_GRID = _T // _TB

_NC, _NS, _L = 2, 16, 16
_NW = _NC * _NS
_PW = (_B * _T) // _NW
_CH = 128
_NCH = _PW // _CH


def _dense_body(qp_ref, ze_ref, emb_ref, out_ref):
    i = pl.program_id(0)
    qp = qp_ref[...]
    ze = ze_ref[...]
    emb = emb_ref[...]

    s1 = jnp.sum(ze, axis=1)
    s2 = jnp.sum(ze * ze, axis=1)

    m = jnp.max(qp, axis=1)
    lse = m + jnp.log(jnp.sum(jnp.exp(qp - m[:, None, :]), axis=1))

    e2q = float(_Q) * emb * emb
    dmin = []
    for b in range(_B):
        d_b = e2q - (2.0 * s1[b])[None, :] * emb
        dmin.append(jnp.min(d_b, axis=0))
    minval = jnp.stack(dmin) + s2

    part = jnp.sum((1.0 + _BETA) * minval - lse)

    @pl.when(i == 0)
    def _():
        out_ref[0, 0] = 0.0

    out_ref[0, 0] += part


def _dense_loss(qp, ze, emb):
    return pl.pallas_call(
        _dense_body,
        grid=(_GRID,),
        in_specs=[
            pl.BlockSpec((_B, _C, _TB), lambda i: (0, 0, i)),
            pl.BlockSpec((_B, _Q, _TB), lambda i: (0, 0, i)),
            pl.BlockSpec((_K, _TB), lambda i: (0, i)),
        ],
        out_specs=pl.BlockSpec(memory_space=pltpu.SMEM),
        out_shape=jax.ShapeDtypeStruct((1, 1), jnp.float32),
    )(qp, ze, emb)


def _sc_gather_body(tw_hbm, qp_hbm, out_hbm, tw_v, idx_v, vals_v, acc_v, sem):
    wid = lax.axis_index("s") * _NC + lax.axis_index("c")
    base = wid * _PW

    pltpu.sync_copy(tw_hbm.at[pl.ds(base, _PW)], tw_v)

    lane = lax.broadcasted_iota(jnp.int32, (_L,), 0)
    for j in range(_PW // _L):
        vid = base + j * _L + lane
        b = lax.shift_right_logical(vid, 11)
        t = lax.bitwise_and(vid, 2047)
        twj = tw_v[pl.ds(j * _L, _L)]
        idx = (lax.shift_left(b, 19)
               + lax.shift_left(twj, 11)
               + t)
        idx_v[j // (_CH // _L), pl.ds((j % (_CH // _L)) * _L, _L)] = idx

    cps = [
        pltpu.make_async_copy(qp_hbm.at[idx_v.at[ch]], vals_v.at[ch], sem)
        for ch in range(_NCH)
    ]
    for cp in cps:
        cp.start()
    for cp in cps:
        cp.wait()

    acc = jnp.zeros((_L,), jnp.float32)
    for ch in range(_NCH):
        for j in range(_CH // _L):
            acc = acc + vals_v[ch, pl.ds(j * _L, _L)]
    acc_v[...] = acc
    pltpu.sync_copy(acc_v, out_hbm.at[wid])


def _sc_gather_sum(tw_flat, qp_flat):
    mesh = plsc.VectorSubcoreMesh(core_axis_name="c", subcore_axis_name="s")
    fn = functools.partial(
        pl.kernel,
        mesh=mesh,
        out_type=jax.ShapeDtypeStruct((_NW, _L), jnp.float32),
        scratch_types=[
            pltpu.VMEM((_PW,), jnp.int32),
            pltpu.VMEM((_NCH, _CH), jnp.int32),
            pltpu.VMEM((_NCH, _CH), jnp.float32),
            pltpu.VMEM((_L,), jnp.float32),
            pltpu.SemaphoreType.DMA,
        ],
    )(_sc_gather_body)
    return fn(tw_flat, qp_flat)


def kernel(quant_pred, target_wav, ze, emb):
    tw_flat = target_wav.astype(jnp.int32).reshape(_B * _T)
    qp_flat = quant_pred.reshape(_B * _C * _T)
    parts = _sc_gather_sum(tw_flat, qp_flat)
    dense = _dense_loss(quant_pred, ze, emb)[0, 0]
    return (dense + jnp.sum(parts)) / float(_B * _T)


# speedup vs baseline: 2.1424x; 1.2618x over previous
import functools

import jax
import jax.numpy as jnp
from jax import lax
from jax.experimental import pallas as pl
from jax.experimental.pallas import tpu as pltpu
from jax.experimental.pallas import tpu_sc as plsc

_BETA = 0.25
_B, _Q, _K, _C, _T = 8, 64, 512, 256, 2048
_TB = 256
_GRID = _T // _TB

_NC, _NS, _L = 2, 16, 16
_NW = _NC * _NS
_TW = 128
_NSLAB = _T // _TW
_BG = _B // 2
_NV = _TW // _L
_KU = 4


def _dense_body(qp_ref, tw_ref, out_ref):
    i = pl.program_id(0)
    qp = qp_ref[...]
    tw = tw_ref[...]

    m = jnp.max(qp, axis=1)
    lse = m + jnp.log(jnp.sum(jnp.exp(qp - m[:, None, :]), axis=1))

    cidx = lax.broadcasted_iota(jnp.int32, (1, _C, 1), 1)
    g = jnp.sum(jnp.where(cidx == tw, qp, 0.0), axis=1)

    part = jnp.sum(g - lse)

    @pl.when(i == 0)
    def _():
        out_ref[0, 0] = 0.0

    out_ref[0, 0] += part


def _softmax_loss(qp, tw):
    return pl.pallas_call(
        _dense_body,
        grid=(_GRID,),
        in_specs=[
            pl.BlockSpec((_B, _C, _TB), lambda i: (0, 0, i)),
            pl.BlockSpec((_B, 1, _TB), lambda i: (0, 0, i)),
        ],
        out_specs=pl.BlockSpec(memory_space=pltpu.SMEM),
        out_shape=jax.ShapeDtypeStruct((1, 1), jnp.float32),
    )(qp, tw)


def _sc_codebook_body(ze_hbm, emb_hbm, out_hbm, ze_v, emb_v, s1_v, acc_v,
                      sem_z, sem_e):
    wid = lax.axis_index("s") * _NC + lax.axis_index("c")
    slab = lax.div(wid, 2)
    t0 = slab * _TW
    b0 = lax.rem(wid, 2) * _BG

    cp_z = pltpu.make_async_copy(
        ze_hbm.at[pl.ds(b0, _BG), :, pl.ds(t0, _TW)], ze_v, sem_z)
    cp_e = pltpu.make_async_copy(
        emb_hbm.at[:, pl.ds(t0, _TW)], emb_v, sem_e)
    cp_z.start()
    cp_e.start()
    cp_z.wait()

    acc = jnp.zeros((_L,), jnp.float32)
    for b in range(_BG):
        def s12_step(q, carry):
            s1s, s2s = carry
            s1o, s2o = [], []
            for v in range(_NV):
                x = ze_v[b, q, pl.ds(v * _L, _L)]
                s1o.append(s1s[v] + x)
                s2o.append(s2s[v] + x * x)
            return tuple(s1o), tuple(s2o)

        zer = tuple(jnp.zeros((_L,), jnp.float32) for _ in range(_NV))
        s1s, s2s = lax.fori_loop(0, _Q, s12_step, (zer, zer))
        for v in range(_NV):
            s1_v[b, pl.ds(v * _L, _L)] = s1s[v]
            acc = acc + s2s[v]

    cp_e.wait()

    for v in range(_NV):
        s1s = [s1_v[b, pl.ds(v * _L, _L)] for b in range(_BG)]

        def min_step(kk, dmins):
            k = kk * _KU
            for u in range(_KU):
                e = emb_v[k + u, pl.ds(v * _L, _L)]
                ne = e * (-2.0)
                e2q = (e * e) * float(_Q)
                dmins = tuple(
                    jnp.minimum(dmins[b], e2q + ne * s1s[b])
                    for b in range(_BG)
                )
            return dmins

        init = tuple(jnp.full((_L,), jnp.inf, jnp.float32)
                     for _ in range(_BG))
        dmins = lax.fori_loop(0, _K // _KU, min_step, init)
        for b in range(_BG):
            acc = acc + dmins[b]

    acc_v[...] = acc
    pltpu.sync_copy(acc_v, out_hbm.at[wid])


def _codebook_loss(ze, emb):
    mesh = plsc.VectorSubcoreMesh(core_axis_name="c", subcore_axis_name="s")
    fn = functools.partial(
        pl.kernel,
        mesh=mesh,
        out_type=jax.ShapeDtypeStruct((_NW, _L), jnp.float32),
        scratch_types=[
            pltpu.VMEM((_BG, _Q, _TW), jnp.float32),
            pltpu.VMEM((_K, _TW), jnp.float32),
            pltpu.VMEM((_BG, _TW), jnp.float32),
            pltpu.VMEM((_L,), jnp.float32),
            pltpu.SemaphoreType.DMA,
            pltpu.SemaphoreType.DMA,
        ],
    )(_sc_codebook_body)
    return fn(ze, emb)


def kernel(quant_pred, target_wav, ze, emb):
    tw = target_wav.astype(jnp.int32)
    parts = _codebook_loss(ze, emb)
    soft = _softmax_loss(quant_pred, tw)[0, 0]
    total = soft + (1.0 + _BETA) * jnp.sum(parts)
    return total / float(_B * _T)
